# baseline (speedup 1.0000x reference)
import jax
import jax.numpy as jnp
from jax.experimental import pallas as pl
from jax.experimental.pallas import tpu as pltpu


def _readout_kernel(x_ref, oh_ref, w_ref, rt_ref, bt_ref, out_ref, wft_ref):
    T = out_ref.shape[0]
    E = out_ref.shape[2]

    @pl.when((pl.program_id(0) == 0) & (pl.program_id(1) == 0))
    def _fold_weights():
        for t in range(T):
            wft_ref[t * E:(t + 1) * E, :] = jnp.transpose(
                w_ref[t], (1, 0)).astype(jnp.bfloat16)

    accT = jax.lax.dot_general(
        wft_ref[...], x_ref[0],
        dimension_numbers=(((1,), (1,)), ((), ())),
        preferred_element_type=jnp.float32)
    maskexT = jnp.dot(rt_ref[...], oh_ref[0],
                      preferred_element_type=jnp.float32)
    maskedT = (accT + bt_ref[...]) * maskexT
    for t in range(T):
        out_ref[t, 0] = maskedT[t * E:(t + 1) * E, :]


def kernel(output_latents, output_task_index, W, b):
    B, N, D = output_latents.shape
    T, _, E = W.shape

    xb = output_latents.astype(jnp.bfloat16)
    onehot = (output_task_index[:, None, :]
              == jnp.arange(T, dtype=output_task_index.dtype)[None, :, None]
              ).astype(jnp.bfloat16)
    tids = jnp.arange(T * E, dtype=jnp.int32) // E
    RT = (tids[:, None] == jnp.arange(T, dtype=jnp.int32)[None, :]).astype(
        jnp.bfloat16)
    bT = b.reshape(T * E, 1)

    TN = 2048
    grid = (B, N // TN)

    out = pl.pallas_call(
        _readout_kernel,
        grid=grid,
        in_specs=[
            pl.BlockSpec((1, TN, D), lambda b_, n: (b_, n, 0)),
            pl.BlockSpec((1, T, TN), lambda b_, n: (b_, 0, n)),
            pl.BlockSpec((T, D, E), lambda b_, n: (0, 0, 0)),
            pl.BlockSpec((T * E, T), lambda b_, n: (0, 0)),
            pl.BlockSpec((T * E, 1), lambda b_, n: (0, 0)),
        ],
        out_specs=pl.BlockSpec((T, 1, E, TN), lambda b_, n: (0, b_, 0, n)),
        out_shape=jax.ShapeDtypeStruct((T, B, E, N), jnp.float32),
        scratch_shapes=[pltpu.VMEM((T * E, D), jnp.bfloat16)],
        compiler_params=pltpu.CompilerParams(
            allow_input_fusion=[True, True, True, True, True]),
    )(xb, onehot, W, RT, bT)
    return jnp.transpose(out, (0, 1, 3, 2))


# speedup vs baseline: 1.5941x; 1.5941x over previous
import jax
import jax.numpy as jnp
from jax.experimental import pallas as pl
from jax.experimental.pallas import tpu as pltpu


def _readout_kernel(x_ref, idx_ref, w_ref, b_ref, out_ref, wft_ref, bt_ref):
    T = out_ref.shape[0]
    E = out_ref.shape[2]
    TN = out_ref.shape[3]

    @pl.when((pl.program_id(0) == 0) & (pl.program_id(1) == 0))
    def _fold_weights():
        for t in range(T):
            wft_ref[t * E:(t + 1) * E, :] = jnp.transpose(
                w_ref[t], (1, 0)).astype(jnp.bfloat16)
            bt_ref[t * E:(t + 1) * E, :] = jnp.transpose(
                b_ref[t:t + 1, :], (1, 0))

    row = idx_ref[pl.ds(pl.program_id(0), 1), :]
    tgrid = jax.lax.broadcasted_iota(jnp.int32, (T, TN), 0)
    oh = (tgrid == row).astype(jnp.bfloat16)
    rt = (jax.lax.broadcasted_iota(jnp.int32, (T * E, T), 0) // E
          == jax.lax.broadcasted_iota(jnp.int32, (T * E, T), 1)
          ).astype(jnp.bfloat16)

    accT = jax.lax.dot_general(
        wft_ref[...], x_ref[0],
        dimension_numbers=(((1,), (1,)), ((), ())),
        preferred_element_type=jnp.float32)
    maskexT = jnp.dot(rt, oh, preferred_element_type=jnp.float32)
    maskedT = (accT + bt_ref[...]) * maskexT
    for t in range(T):
        out_ref[t, 0] = maskedT[t * E:(t + 1) * E, :]


def kernel(output_latents, output_task_index, W, b):
    B, N, D = output_latents.shape
    T, _, E = W.shape

    xb = output_latents.astype(jnp.bfloat16)

    TN = 1024
    grid = (B, N // TN)

    out = pl.pallas_call(
        _readout_kernel,
        grid=grid,
        in_specs=[
            pl.BlockSpec((1, TN, D), lambda b_, n: (b_, n, 0)),
            pl.BlockSpec((B, TN), lambda b_, n: (0, n)),
            pl.BlockSpec((T, D, E), lambda b_, n: (0, 0, 0)),
            pl.BlockSpec((T, E), lambda b_, n: (0, 0)),
        ],
        out_specs=pl.BlockSpec((T, 1, E, TN), lambda b_, n: (0, b_, 0, n)),
        out_shape=jax.ShapeDtypeStruct((T, B, E, N), jnp.float32),
        scratch_shapes=[pltpu.VMEM((T * E, D), jnp.bfloat16),
                        pltpu.VMEM((T * E, 1), jnp.float32)],
        compiler_params=pltpu.CompilerParams(
            allow_input_fusion=[True, True, True, True]),
    )(xb, output_task_index, W, b)
    return jnp.transpose(out, (0, 1, 3, 2))


# speedup vs baseline: 1.7172x; 1.0772x over previous
import jax
import jax.numpy as jnp
from jax.experimental import pallas as pl
from jax.experimental.pallas import tpu as pltpu


def _readout_kernel(x_ref, idx_ref, wf_ref, b_ref, out_ref, bt_ref):
    T = out_ref.shape[0]
    E = out_ref.shape[2]
    TN = out_ref.shape[3]

    @pl.when((pl.program_id(0) == 0) & (pl.program_id(1) == 0))
    def _fold_bias():
        for t in range(T):
            bt_ref[t * E:(t + 1) * E, :] = jnp.transpose(
                b_ref[t:t + 1, :], (1, 0))

    row = idx_ref[pl.ds(pl.program_id(0), 1), :]
    tgrid = jax.lax.broadcasted_iota(jnp.int32, (T, TN), 0)
    oh = (tgrid == row).astype(jnp.bfloat16)
    rt = (jax.lax.broadcasted_iota(jnp.int32, (T * E, T), 0) // E
          == jax.lax.broadcasted_iota(jnp.int32, (T * E, T), 1)
          ).astype(jnp.bfloat16)

    accT = jax.lax.dot_general(
        wf_ref[...], x_ref[0],
        dimension_numbers=(((1,), (1,)), ((), ())),
        preferred_element_type=jnp.float32)
    maskexT = jnp.dot(rt, oh, preferred_element_type=jnp.float32)
    maskedT = (accT + bt_ref[...]) * maskexT
    for t in range(T):
        out_ref[t, 0] = maskedT[t * E:(t + 1) * E, :]


def kernel(output_latents, output_task_index, W, b):
    B, N, D = output_latents.shape
    T, _, E = W.shape

    xb = output_latents.astype(jnp.bfloat16)
    Wfold = jnp.transpose(W, (0, 2, 1)).reshape(T * E, D).astype(jnp.bfloat16)

    TN = 1024
    grid = (B, N // TN)

    out = pl.pallas_call(
        _readout_kernel,
        grid=grid,
        in_specs=[
            pl.BlockSpec((1, TN, D), lambda b_, n: (b_, n, 0)),
            pl.BlockSpec((B, TN), lambda b_, n: (0, n)),
            pl.BlockSpec((T * E, D), lambda b_, n: (0, 0)),
            pl.BlockSpec((T, E), lambda b_, n: (0, 0)),
        ],
        out_specs=pl.BlockSpec((T, 1, E, TN), lambda b_, n: (0, b_, 0, n)),
        out_shape=jax.ShapeDtypeStruct((T, B, E, N), jnp.float32),
        scratch_shapes=[pltpu.VMEM((T * E, 1), jnp.float32)],
        compiler_params=pltpu.CompilerParams(
            allow_input_fusion=[True, True, True, True]),
    )(xb, output_task_index, Wfold, b)
    return jnp.transpose(out, (0, 1, 3, 2))
